# baseline (speedup 1.0000x reference)
import jax
import jax.numpy as jnp
from jax import lax
from jax.experimental import pallas as pl
from jax.experimental.pallas import tpu as pltpu

V = 100000
D = 128
C = 20
N = 128
VB = 32768
NBLK = (V + VB - 1) // VB
PADV = NBLK * VB


def _fused_kernel(idx_ref, emb_ref, w1_ref, b1_ref, w2_ref, b2_ref,
                  out_ref, g_ref, h_ref, sem):
    i = pl.program_id(0)

    @pl.when(i == 0)
    def _gather_and_hidden():
        for p in range(C):
            pltpu.make_async_copy(
                emb_ref.at[pl.ds(idx_ref[p], 1), :],
                g_ref.at[pl.ds(p, 1), :],
                sem,
            ).start()
        for p in range(C):
            pltpu.make_async_copy(
                emb_ref.at[pl.ds(idx_ref[p], 1), :],
                g_ref.at[pl.ds(p, 1), :],
                sem,
            ).wait()
        acc = b1_ref[...].astype(jnp.float32)
        for p in range(C):
            acc = acc + lax.dot_general(
                g_ref[pl.ds(p, 1), :],
                w1_ref[:, pl.ds(p * D, D)],
                (((1,), (1,)), ((), ())),
                preferred_element_type=jnp.float32,
            )
        h_ref[...] = jnp.maximum(acc, 0.0)

    lb = lax.dot_general(
        h_ref[...],
        w2_ref[...],
        (((1,), (1,)), ((), ())),
        preferred_element_type=jnp.float32,
    ) + b2_ref[...]

    @pl.when(i < NBLK - 1)
    def _store():
        out_ref[0:1, pl.ds(i * VB, VB)] = lb

    @pl.when(i == NBLK - 1)
    def _store_masked_and_normalize():
        cols = (NBLK - 1) * VB + lax.broadcasted_iota(jnp.int32, (1, VB), 1)
        out_ref[0:1, pl.ds((NBLK - 1) * VB, VB)] = jnp.where(cols < V, lb, -1e30)
        scr = out_ref[...]
        m = jnp.max(scr, axis=1, keepdims=True)
        s = jnp.sum(jnp.exp(scr - m), axis=1, keepdims=True)
        out_ref[...] = scr - (m + jnp.log(s))


def kernel(inputs, emb, W1, b1, W2, b2):
    b1r = b1.reshape(1, N)
    b2r = b2.reshape(1, V)
    out = pl.pallas_call(
        _fused_kernel,
        grid=(NBLK,),
        in_specs=[
            pl.BlockSpec(memory_space=pltpu.MemorySpace.SMEM),
            pl.BlockSpec(memory_space=pltpu.MemorySpace.HBM),
            pl.BlockSpec((N, C * D), lambda i: (0, 0)),
            pl.BlockSpec((1, N), lambda i: (0, 0)),
            pl.BlockSpec((VB, D), lambda i: (i, 0)),
            pl.BlockSpec((1, VB), lambda i: (0, i)),
        ],
        out_specs=pl.BlockSpec((1, PADV), lambda i: (0, 0)),
        out_shape=jax.ShapeDtypeStruct((1, PADV), jnp.float32),
        scratch_shapes=[
            pltpu.VMEM((C, D), jnp.float32),
            pltpu.VMEM((1, N), jnp.float32),
            pltpu.SemaphoreType.DMA,
        ],
        compiler_params=pltpu.CompilerParams(
            dimension_semantics=("arbitrary",),
            vmem_limit_bytes=100 * 1024 * 1024,
        ),
    )(inputs, emb, W1, b1r, W2, b2r)
    return out[:, :V]


# speedup vs baseline: 1.1576x; 1.1576x over previous
import jax
import jax.numpy as jnp
from jax import lax
from jax.experimental import pallas as pl
from jax.experimental.pallas import tpu as pltpu

V = 100000
D = 128
C = 20
N = 128
VB = 20480
NBLK = (V + VB - 1) // VB
PADV = NBLK * VB


def _fused_kernel(idx_ref, emb_ref, w1_ref, b1_ref, w2_ref, b2_ref,
                  out_ref, g_ref, h_ref, sem):
    i = pl.program_id(0)

    @pl.when(i == 0)
    def _gather_and_hidden():
        for p in range(C):
            pltpu.make_async_copy(
                emb_ref.at[pl.ds(idx_ref[p], 1), :],
                g_ref.at[pl.ds(p, 1), :],
                sem,
            ).start()
        for p in range(C):
            pltpu.make_async_copy(
                emb_ref.at[pl.ds(idx_ref[p], 1), :],
                g_ref.at[pl.ds(p, 1), :],
                sem,
            ).wait()
        acc = b1_ref[...].astype(jnp.float32)
        for p in range(C):
            acc = acc + lax.dot_general(
                g_ref[pl.ds(p, 1), :],
                w1_ref[:, pl.ds(p * D, D)],
                (((1,), (1,)), ((), ())),
                preferred_element_type=jnp.float32,
            )
        h_ref[...] = jnp.maximum(acc, 0.0)

    lb = lax.dot_general(
        h_ref[...],
        w2_ref[...],
        (((1,), (1,)), ((), ())),
        preferred_element_type=jnp.float32,
    ) + b2_ref[...]

    @pl.when(i < NBLK - 1)
    def _store():
        out_ref[0:1, pl.ds(i * VB, VB)] = lb

    @pl.when(i == NBLK - 1)
    def _store_masked_and_normalize():
        cols = (NBLK - 1) * VB + lax.broadcasted_iota(jnp.int32, (1, VB), 1)
        out_ref[0:1, pl.ds((NBLK - 1) * VB, VB)] = jnp.where(cols < V, lb, -1e30)
        scr = out_ref[...]
        m = jnp.max(scr, axis=1, keepdims=True)
        s = jnp.sum(jnp.exp(scr - m), axis=1, keepdims=True)
        out_ref[...] = scr - (m + jnp.log(s))


def kernel(inputs, emb, W1, b1, W2, b2):
    b1r = b1.reshape(1, N)
    b2r = b2.reshape(1, V)
    out = pl.pallas_call(
        _fused_kernel,
        grid=(NBLK,),
        in_specs=[
            pl.BlockSpec(memory_space=pltpu.MemorySpace.SMEM),
            pl.BlockSpec(memory_space=pltpu.MemorySpace.HBM),
            pl.BlockSpec((N, C * D), lambda i: (0, 0)),
            pl.BlockSpec((1, N), lambda i: (0, 0)),
            pl.BlockSpec((VB, D), lambda i: (i, 0)),
            pl.BlockSpec((1, VB), lambda i: (0, i)),
        ],
        out_specs=pl.BlockSpec((1, PADV), lambda i: (0, 0)),
        out_shape=jax.ShapeDtypeStruct((1, PADV), jnp.float32),
        scratch_shapes=[
            pltpu.VMEM((C, D), jnp.float32),
            pltpu.VMEM((1, N), jnp.float32),
            pltpu.SemaphoreType.DMA,
        ],
        compiler_params=pltpu.CompilerParams(
            dimension_semantics=("arbitrary",),
            vmem_limit_bytes=100 * 1024 * 1024,
        ),
    )(inputs, emb, W1, b1r, W2, b2r)
    return out[:, :V]


# speedup vs baseline: 1.1697x; 1.0105x over previous
import jax
import jax.numpy as jnp
from jax import lax
from jax.experimental import pallas as pl
from jax.experimental.pallas import tpu as pltpu

V = 100000
D = 128
C = 20
N = 128
VB = 12800
NBLK = (V + VB - 1) // VB
PADV = NBLK * VB


def _fused_kernel(idx_ref, emb_ref, w1_ref, b1_ref, w2_ref, b2_ref,
                  out_ref, g_ref, h_ref, sem):
    i = pl.program_id(0)

    @pl.when(i == 0)
    def _gather_and_hidden():
        for p in range(C):
            pltpu.make_async_copy(
                emb_ref.at[pl.ds(idx_ref[p], 1), :],
                g_ref.at[pl.ds(p, 1), :],
                sem,
            ).start()
        for p in range(C):
            pltpu.make_async_copy(
                emb_ref.at[pl.ds(idx_ref[p], 1), :],
                g_ref.at[pl.ds(p, 1), :],
                sem,
            ).wait()
        acc = b1_ref[...].astype(jnp.float32)
        for p in range(C):
            acc = acc + lax.dot_general(
                g_ref[pl.ds(p, 1), :],
                w1_ref[:, pl.ds(p * D, D)],
                (((1,), (1,)), ((), ())),
                preferred_element_type=jnp.float32,
            )
        h_ref[...] = jnp.maximum(acc, 0.0)

    lb = lax.dot_general(
        h_ref[...],
        w2_ref[...],
        (((1,), (1,)), ((), ())),
        preferred_element_type=jnp.float32,
    ) + b2_ref[...]

    @pl.when(i < NBLK - 1)
    def _store():
        out_ref[0:1, pl.ds(i * VB, VB)] = lb

    @pl.when(i == NBLK - 1)
    def _store_masked_and_normalize():
        cols = (NBLK - 1) * VB + lax.broadcasted_iota(jnp.int32, (1, VB), 1)
        out_ref[0:1, pl.ds((NBLK - 1) * VB, VB)] = jnp.where(cols < V, lb, -1e30)
        scr = out_ref[...]
        m = jnp.max(scr, axis=1, keepdims=True)
        s = jnp.sum(jnp.exp(scr - m), axis=1, keepdims=True)
        out_ref[...] = scr - (m + jnp.log(s))


def kernel(inputs, emb, W1, b1, W2, b2):
    b1r = b1.reshape(1, N)
    b2r = b2.reshape(1, V)
    out = pl.pallas_call(
        _fused_kernel,
        grid=(NBLK,),
        in_specs=[
            pl.BlockSpec(memory_space=pltpu.MemorySpace.SMEM),
            pl.BlockSpec(memory_space=pltpu.MemorySpace.HBM),
            pl.BlockSpec((N, C * D), lambda i: (0, 0)),
            pl.BlockSpec((1, N), lambda i: (0, 0)),
            pl.BlockSpec((VB, D), lambda i: (i, 0)),
            pl.BlockSpec((1, VB), lambda i: (0, i)),
        ],
        out_specs=pl.BlockSpec((1, PADV), lambda i: (0, 0)),
        out_shape=jax.ShapeDtypeStruct((1, PADV), jnp.float32),
        scratch_shapes=[
            pltpu.VMEM((C, D), jnp.float32),
            pltpu.VMEM((1, N), jnp.float32),
            pltpu.SemaphoreType.DMA,
        ],
        compiler_params=pltpu.CompilerParams(
            dimension_semantics=("arbitrary",),
            vmem_limit_bytes=100 * 1024 * 1024,
        ),
    )(inputs, emb, W1, b1r, W2, b2r)
    return out[:, :V]
